# Initial kernel scaffold; baseline (speedup 1.0000x reference)
import functools

import jax
import jax.numpy as jnp
from jax import lax
from jax.experimental import pallas as pl
from jax.experimental.pallas import tpu as pltpu
from jax.experimental.pallas import tpu_sc as plsc

EMBED_ = 64
SEQ_ = 50
BATCH_ = 1024
SAMPLE_ = 20

_NC = 2
_NS = 16
_NW = _NC * _NS


def _make_sc_gather(n_rows: int, chunk: int, embed: int):
    assert n_rows % _NW == 0
    rows_per_worker = n_rows // _NW
    assert rows_per_worker % chunk == 0
    n_chunks = rows_per_worker // chunk
    assert chunk % 8 == 0

    mesh = plsc.VectorSubcoreMesh(core_axis_name="c", subcore_axis_name="s")

    @functools.partial(
        pl.kernel,
        mesh=mesh,
        out_type=jax.ShapeDtypeStruct((n_rows, embed), jnp.float32),
        scratch_types=[
            pltpu.VMEM((chunk,), jnp.int32),
            pltpu.VMEM((chunk, embed), jnp.float32),
            pltpu.SemaphoreType.DMA,
        ],
    )
    def gather(table_hbm, idx_hbm, out_hbm, idx_v, rows_v, sem):
        wid = lax.axis_index("s") * _NC + lax.axis_index("c")
        base = wid * rows_per_worker

        def body(j, carry):
            off = base + j * chunk
            pltpu.sync_copy(idx_hbm.at[pl.ds(off, chunk)], idx_v)
            pltpu.async_copy(table_hbm.at[idx_v], rows_v, sem).wait()
            pltpu.sync_copy(rows_v, out_hbm.at[pl.ds(off, chunk)])
            return carry

        lax.fori_loop(0, n_chunks, body, 0)

    return gather


def _lstm_step(x_ref, wih_ref, whh_ref, b_ref, out_ref, h_ref, c_ref):
    t = pl.program_id(0)

    @pl.when(t == 0)
    def _init():
        h_ref[...] = jnp.zeros_like(h_ref)
        c_ref[...] = jnp.zeros_like(c_ref)

    x = x_ref[0]
    gates = (
        jnp.dot(x, wih_ref[...], preferred_element_type=jnp.float32)
        + jnp.dot(h_ref[...], whh_ref[...], preferred_element_type=jnp.float32)
        + b_ref[...]
    )
    E = h_ref.shape[-1]
    i = jax.nn.sigmoid(gates[:, :E])
    f = jax.nn.sigmoid(gates[:, E:2 * E])
    g = jnp.tanh(gates[:, 2 * E:3 * E])
    o = jax.nn.sigmoid(gates[:, 3 * E:])
    c = f * c_ref[...] + i * g
    h = o * jnp.tanh(c)
    c_ref[...] = c
    h_ref[...] = h
    out_ref[0] = h


def _lstm(x, wih_t, whh_t, b):
    T, B, E = x.shape
    return pl.pallas_call(
        _lstm_step,
        grid=(T,),
        in_specs=[
            pl.BlockSpec((1, B, E), lambda t: (t, 0, 0)),
            pl.BlockSpec((E, 4 * E), lambda t: (0, 0)),
            pl.BlockSpec((E, 4 * E), lambda t: (0, 0)),
            pl.BlockSpec((1, 4 * E), lambda t: (0, 0)),
        ],
        out_specs=pl.BlockSpec((1, B, E), lambda t: (t, 0, 0)),
        out_shape=jax.ShapeDtypeStruct((T, B, E), jnp.float32),
        scratch_shapes=[
            pltpu.VMEM((B, E), jnp.float32),
            pltpu.VMEM((B, E), jnp.float32),
        ],
    )(x, wih_t, whh_t, b)


_gather_text = _make_sc_gather(SEQ_ * BATCH_, 1600, EMBED_)
_gather_out = _make_sc_gather(SEQ_ * BATCH_ * (SAMPLE_ + 1), 1600, EMBED_)


def kernel(samples, text, targets, in_embed, out_embed, W_ih, W_hh, b_ih, b_hh):
    E = in_embed.shape[1]
    T, B, K = samples.shape

    text_emb = _gather_text(in_embed, text.reshape(-1).astype(jnp.int32))
    hs = _lstm(
        text_emb.reshape(T, B, E),
        W_ih.T,
        W_hh.T,
        (b_ih + b_hh).reshape(1, 4 * E),
    )
    rnn_output = hs.reshape(-1, E)[:, :, None]

    idx2 = jnp.concatenate(
        [samples.reshape(-1), targets.reshape(-1)]
    ).astype(jnp.int32)
    rows = _gather_out(out_embed, idx2)
    samples_embedded = rows[: T * B * K].reshape(T * B, K, E)
    targets_embedded = rows[T * B * K:].reshape(T * B, 1, E)
    return samples_embedded, rnn_output, targets_embedded


# speedup vs baseline: 2.7091x; 2.7091x over previous
import functools

import jax
import jax.numpy as jnp
from jax import lax
from jax.experimental import pallas as pl
from jax.experimental.pallas import tpu as pltpu
from jax.experimental.pallas import tpu_sc as plsc

EMBED_ = 64
SEQ_ = 50
BATCH_ = 1024
SAMPLE_ = 20

_NC = 2
_NS = 16
_NW = _NC * _NS


def _make_sc_gather(n_rows: int, chunk: int, embed: int):
    assert n_rows % _NW == 0
    rows_per_worker = n_rows // _NW
    assert rows_per_worker % chunk == 0
    n_chunks = rows_per_worker // chunk
    assert chunk % 8 == 0

    mesh = plsc.VectorSubcoreMesh(core_axis_name="c", subcore_axis_name="s")

    @functools.partial(
        pl.kernel,
        mesh=mesh,
        out_type=jax.ShapeDtypeStruct((n_rows, embed), jnp.float32),
        scratch_types=[
            pltpu.VMEM((chunk,), jnp.int32),
            pltpu.VMEM((chunk, embed), jnp.float32),
            pltpu.SemaphoreType.DMA,
        ],
        compiler_params=pltpu.CompilerParams(use_tc_tiling_on_sc=False),
    )
    def gather(table_hbm, idx_hbm, out_hbm, idx_v, rows_v, sem):
        wid = lax.axis_index("s") * _NC + lax.axis_index("c")
        base = wid * rows_per_worker

        def body(j, carry):
            off = base + j * chunk
            pltpu.sync_copy(idx_hbm.at[pl.ds(off, chunk)], idx_v)
            pltpu.async_copy(table_hbm.at[idx_v], rows_v, sem).wait()
            pltpu.sync_copy(rows_v, out_hbm.at[pl.ds(off, chunk)])
            return carry

        lax.fori_loop(0, n_chunks, body, 0)

    return gather


def _lstm_step(x_ref, wih_ref, whh_ref, b_ref, out_ref, h_ref, c_ref):
    t = pl.program_id(0)

    @pl.when(t == 0)
    def _init():
        h_ref[...] = jnp.zeros_like(h_ref)
        c_ref[...] = jnp.zeros_like(c_ref)

    x = x_ref[0]
    gates = (
        jnp.dot(x, wih_ref[...], preferred_element_type=jnp.float32)
        + jnp.dot(h_ref[...], whh_ref[...], preferred_element_type=jnp.float32)
        + b_ref[...]
    )
    E = h_ref.shape[-1]
    i = jax.nn.sigmoid(gates[:, :E])
    f = jax.nn.sigmoid(gates[:, E:2 * E])
    g = jnp.tanh(gates[:, 2 * E:3 * E])
    o = jax.nn.sigmoid(gates[:, 3 * E:])
    c = f * c_ref[...] + i * g
    h = o * jnp.tanh(c)
    c_ref[...] = c
    h_ref[...] = h
    out_ref[0] = h


def _lstm(x, wih_t, whh_t, b):
    T, B, E = x.shape
    return pl.pallas_call(
        _lstm_step,
        grid=(T,),
        in_specs=[
            pl.BlockSpec((1, B, E), lambda t: (t, 0, 0)),
            pl.BlockSpec((E, 4 * E), lambda t: (0, 0)),
            pl.BlockSpec((E, 4 * E), lambda t: (0, 0)),
            pl.BlockSpec((1, 4 * E), lambda t: (0, 0)),
        ],
        out_specs=pl.BlockSpec((1, B, E), lambda t: (t, 0, 0)),
        out_shape=jax.ShapeDtypeStruct((T, B, E), jnp.float32),
        scratch_shapes=[
            pltpu.VMEM((B, E), jnp.float32),
            pltpu.VMEM((B, E), jnp.float32),
        ],
    )(x, wih_t, whh_t, b)


_gather_text = _make_sc_gather(SEQ_ * BATCH_, 1600, EMBED_)
_gather_out = _make_sc_gather(SEQ_ * BATCH_ * (SAMPLE_ + 1), 1600, EMBED_)


def kernel(samples, text, targets, in_embed, out_embed, W_ih, W_hh, b_ih, b_hh):
    E = in_embed.shape[1]
    T, B, K = samples.shape

    text_emb = _gather_text(in_embed, text.reshape(-1).astype(jnp.int32))
    hs = _lstm(
        text_emb.reshape(T, B, E),
        W_ih.T,
        W_hh.T,
        (b_ih + b_hh).reshape(1, 4 * E),
    )
    rnn_output = hs.reshape(-1, E)[:, :, None]

    idx2 = jnp.concatenate(
        [samples.reshape(-1), targets.reshape(-1)]
    ).astype(jnp.int32)
    rows = _gather_out(out_embed, idx2)
    samples_embedded = rows[: T * B * K].reshape(T * B, K, E)
    targets_embedded = rows[T * B * K:].reshape(T * B, 1, E)
    return samples_embedded, rnn_output, targets_embedded


# speedup vs baseline: 4.8680x; 1.7969x over previous
import functools

import jax
import jax.numpy as jnp
from jax import lax
from jax.experimental import pallas as pl
from jax.experimental.pallas import tpu as pltpu
from jax.experimental.pallas import tpu_sc as plsc

EMBED_ = 64
SEQ_ = 50
BATCH_ = 1024
SAMPLE_ = 20

_NC = 2
_NS = 16
_NW = _NC * _NS


def _make_sc_gather(n_rows: int, chunk: int, embed: int):
    assert n_rows % _NW == 0
    rows_per_worker = n_rows // _NW
    assert rows_per_worker % chunk == 0
    n_chunks = rows_per_worker // chunk
    assert chunk % 8 == 0

    assert n_chunks == 1 or n_chunks % 2 == 0
    nbuf = 1 if n_chunks == 1 else 2
    mesh = plsc.VectorSubcoreMesh(core_axis_name="c", subcore_axis_name="s")

    @functools.partial(
        pl.kernel,
        mesh=mesh,
        out_type=jax.ShapeDtypeStruct((n_rows, embed), jnp.float32),
        scratch_types=[
            pltpu.VMEM((nbuf, chunk), jnp.int32),
            pltpu.VMEM((nbuf, chunk, embed), jnp.float32),
            pltpu.SemaphoreType.DMA,
            pltpu.SemaphoreType.DMA,
            pltpu.SemaphoreType.DMA,
            pltpu.SemaphoreType.DMA,
        ],
        compiler_params=pltpu.CompilerParams(use_tc_tiling_on_sc=False),
    )
    def gather(table_hbm, idx_hbm, out_hbm, idx_v, rows_v,
               g_sem0, g_sem1, o_sem0, o_sem1):
        wid = lax.axis_index("s") * _NC + lax.axis_index("c")
        base = wid * rows_per_worker
        g_sems = (g_sem0, g_sem1)
        o_sems = (o_sem0, o_sem1)

        def start_gather(j, b):
            pltpu.sync_copy(idx_hbm.at[pl.ds(base + j * chunk, chunk)],
                            idx_v.at[b])
            pltpu.async_copy(table_hbm.at[idx_v.at[b]], rows_v.at[b],
                             g_sems[b])

        def wait_gather(b):
            pltpu.make_async_copy(table_hbm.at[idx_v.at[b]], rows_v.at[b],
                                  g_sems[b]).wait()

        def out_copy_obj(j, b):
            return pltpu.make_async_copy(
                rows_v.at[b], out_hbm.at[pl.ds(base + j * chunk, chunk)],
                o_sems[b])

        if n_chunks == 1:
            start_gather(0, 0)
            wait_gather(0)
            pltpu.sync_copy(rows_v.at[0], out_hbm.at[pl.ds(base, chunk)])
            return

        start_gather(0, 0)
        start_gather(1, 1)

        def body(jj, carry):
            for b in (0, 1):
                j = 2 * jj + b
                wait_gather(b)
                out_copy_obj(j, b).start()

                @pl.when(j + 2 < n_chunks)
                def _next():
                    out_copy_obj(j, b).wait()
                    start_gather(j + 2, b)

            return carry

        lax.fori_loop(0, n_chunks // 2, body, 0)
        out_copy_obj(n_chunks - 2, 0).wait()
        out_copy_obj(n_chunks - 1, 1).wait()

    return gather


def _lstm_step(x_ref, wih_ref, whh_ref, b_ref, out_ref, h_ref, c_ref):
    t = pl.program_id(0)

    @pl.when(t == 0)
    def _init():
        h_ref[...] = jnp.zeros_like(h_ref)
        c_ref[...] = jnp.zeros_like(c_ref)

    x = x_ref[0]
    gates = (
        jnp.dot(x, wih_ref[...], preferred_element_type=jnp.float32)
        + jnp.dot(h_ref[...], whh_ref[...], preferred_element_type=jnp.float32)
        + b_ref[...]
    )
    E = h_ref.shape[-1]
    i = jax.nn.sigmoid(gates[:, :E])
    f = jax.nn.sigmoid(gates[:, E:2 * E])
    g = jnp.tanh(gates[:, 2 * E:3 * E])
    o = jax.nn.sigmoid(gates[:, 3 * E:])
    c = f * c_ref[...] + i * g
    h = o * jnp.tanh(c)
    c_ref[...] = c
    h_ref[...] = h
    out_ref[0] = h


def _lstm(x, wih_t, whh_t, b):
    T, B, E = x.shape
    return pl.pallas_call(
        _lstm_step,
        grid=(T,),
        in_specs=[
            pl.BlockSpec((1, B, E), lambda t: (t, 0, 0)),
            pl.BlockSpec((E, 4 * E), lambda t: (0, 0)),
            pl.BlockSpec((E, 4 * E), lambda t: (0, 0)),
            pl.BlockSpec((1, 4 * E), lambda t: (0, 0)),
        ],
        out_specs=pl.BlockSpec((1, B, E), lambda t: (t, 0, 0)),
        out_shape=jax.ShapeDtypeStruct((T, B, E), jnp.float32),
        scratch_shapes=[
            pltpu.VMEM((B, E), jnp.float32),
            pltpu.VMEM((B, E), jnp.float32),
        ],
    )(x, wih_t, whh_t, b)


_gather_text = _make_sc_gather(SEQ_ * BATCH_, 1600, EMBED_)
_gather_samples = _make_sc_gather(SEQ_ * BATCH_ * SAMPLE_, 800, EMBED_)
_gather_targets = _make_sc_gather(SEQ_ * BATCH_, 1600, EMBED_)


def kernel(samples, text, targets, in_embed, out_embed, W_ih, W_hh, b_ih, b_hh):
    E = in_embed.shape[1]
    T, B, K = samples.shape

    text_emb = _gather_text(in_embed, text.reshape(-1).astype(jnp.int32))
    hs = _lstm(
        text_emb.reshape(T, B, E),
        W_ih.T,
        W_hh.T,
        (b_ih + b_hh).reshape(1, 4 * E),
    )
    rnn_output = hs.reshape(-1, E)[:, :, None]

    rows_s = _gather_samples(out_embed, samples.reshape(-1).astype(jnp.int32))
    rows_t = _gather_targets(out_embed, targets.reshape(-1).astype(jnp.int32))
    samples_embedded = rows_s.reshape(T * B, K, E)
    targets_embedded = rows_t.reshape(T * B, 1, E)
    return samples_embedded, rnn_output, targets_embedded
